# Initial kernel scaffold; baseline (speedup 1.0000x reference)
import jax
import jax.numpy as jnp
from jax.experimental import pallas as pl


def kernel(indices, offsets, tt_core0, tt_core1, tt_core2):
    raise NotImplementedError("write your pallas kernel here")


# speedup vs baseline: 54.7225x; 54.7225x over previous
import jax
import jax.numpy as jnp
from jax import lax
from jax.experimental import pallas as pl
from jax.experimental.pallas import tpu as pltpu, tpu_sc as plsc
import functools

_T = 26
_B = 1024
_L = 20
_D = 64
_P = (100, 100, 100)
_NNZ = _T * _B * _L
_NW = 32
_PER_W = _NNZ // _NW
_BAGS_W = _PER_W // _L
_CHUNK = 80
_NCHUNK = _PER_W // _CHUNK
_BPC = _CHUNK // _L


def _sc_body(idx_hbm, c0_hbm, c1_hbm, c2_hbm, out_hbm,
             idx_v, f0_v, f1_v, f2_v, a_v, b_v, c_v, res_v, ob_v,
             sem0, sem1, sem2):
    wid = lax.axis_index("s") * 2 + lax.axis_index("c")
    base = wid * _PER_W
    obase = wid * _BAGS_W * _D
    lanes = lax.iota(jnp.int32, 16)
    pats = [lanes * 8 + r2 for r2 in range(8)]

    def chunk_body(g, carry):
        cbase = base + g * _CHUNK
        pltpu.sync_copy(idx_hbm.at[pl.ds(cbase, _CHUNK)], idx_v)
        for v in range(_CHUNK // 16):
            ii = idx_v[pl.ds(v * 16, 16)]
            t100 = ((cbase + v * 16 + lanes) // (_B * _L)) * _P[0]
            f0_v[pl.ds(v * 16, 16)] = t100 + ii // (_P[1] * _P[2])
            f1_v[pl.ds(v * 16, 16)] = t100 + (ii // _P[2]) % _P[1]
            f2_v[pl.ds(v * 16, 16)] = t100 + ii % _P[2]
        cp0 = pltpu.async_copy(c0_hbm.at[f0_v], a_v, sem0)
        cp1 = pltpu.async_copy(c1_hbm.at[f1_v], b_v, sem1)
        cp2 = pltpu.async_copy(c2_hbm.at[f2_v], c_v, sem2)
        cp0.wait()
        cp1.wait()
        cp2.wait()

        def bag_body(bb, carry2):
            def lk_body(l, accs):
                li = bb * _L + l
                av = [a_v[li, pl.ds(16 * k, 16)] for k in range(2)]
                cv = [c_v[li, pl.ds(16 * k, 16)] for k in range(2)]
                bv = [b_v[li, pl.ds(16 * k, 16)] for k in range(16)]
                for j in range(8):
                    q0, h = j // 2, j % 2
                    r = av[q0 // 2][(q0 * 8) % 16] * bv[h]
                    for r1 in range(1, 8):
                        p = q0 * 8 + r1
                        r = r + av[p // 16][p % 16] * bv[r1 * 2 + h]
                    res_v[pl.ds(16 * j, 16)] = r
                a0, a1, a2, a3 = accs
                for r2 in range(8):
                    gv = plsc.load_gather(res_v, [pats[r2]])
                    a0 = a0 + gv * cv[(r2 * 4) // 16][(r2 * 4) % 16]
                    a1 = a1 + gv * cv[(r2 * 4 + 1) // 16][(r2 * 4 + 1) % 16]
                    a2 = a2 + gv * cv[(r2 * 4 + 2) // 16][(r2 * 4 + 2) % 16]
                    a3 = a3 + gv * cv[(r2 * 4 + 3) // 16][(r2 * 4 + 3) % 16]
                return (a0, a1, a2, a3)

            z = jnp.zeros((16,), jnp.float32)
            accs = lax.fori_loop(0, _L, lk_body, (z, z, z, z))
            for m in range(4):
                plsc.store_scatter(ob_v, [bb * _D + lanes * 4 + m], accs[m])
            return carry2

        lax.fori_loop(0, _BPC, bag_body, 0)
        pltpu.sync_copy(ob_v, out_hbm.at[pl.ds(obase + g * _BPC * _D,
                                               _BPC * _D)])
        return carry

    lax.fori_loop(0, _NCHUNK, chunk_body, 0)


@jax.jit
def _tt_bag_sc(indices, c0f, c1f, c2f):
    mesh = plsc.VectorSubcoreMesh(core_axis_name="c", subcore_axis_name="s")
    f = pl.kernel(
        _sc_body,
        out_type=jax.ShapeDtypeStruct((_T * _B * _D,), jnp.float32),
        mesh=mesh,
        compiler_params=pltpu.CompilerParams(needs_layout_passes=False,
                                             use_tc_tiling_on_sc=False),
        scratch_types=[
            pltpu.VMEM((_CHUNK,), jnp.int32),
            pltpu.VMEM((_CHUNK,), jnp.int32),
            pltpu.VMEM((_CHUNK,), jnp.int32),
            pltpu.VMEM((_CHUNK,), jnp.int32),
            pltpu.VMEM((_CHUNK, 32), jnp.float32),
            pltpu.VMEM((_CHUNK, 256), jnp.float32),
            pltpu.VMEM((_CHUNK, 32), jnp.float32),
            pltpu.VMEM((128,), jnp.float32),
            pltpu.VMEM((_BPC * _D,), jnp.float32),
            pltpu.SemaphoreType.DMA,
            pltpu.SemaphoreType.DMA,
            pltpu.SemaphoreType.DMA,
        ],
    )
    return f(indices, c0f, c1f, c2f)


def kernel(indices, offsets, tt_core0, tt_core1, tt_core2):
    del offsets
    c0f = tt_core0.reshape(_T * _P[0], 32)
    c1f = tt_core1.reshape(_T * _P[1], 256)
    c2f = tt_core2.reshape(_T * _P[2], 32)
    out = _tt_bag_sc(indices, c0f, c1f, c2f)
    return out.reshape(_T * _B, _D)


# speedup vs baseline: 75.2969x; 1.3760x over previous
import jax
import jax.numpy as jnp
from jax import lax
from jax.experimental import pallas as pl
from jax.experimental.pallas import tpu as pltpu, tpu_sc as plsc

_T = 26
_B = 1024
_L = 20
_D = 64
_P = (100, 100, 100)
_NNZ = _T * _B * _L
_NW = 32
_PER_W = _NNZ // _NW
_BAGS_W = _PER_W // _L
_CHUNK = 80
_NCHUNK = _PER_W // _CHUNK
_BPC = _CHUNK // _L


def _sc_body(idx_hbm, c0_hbm, c1_hbm, c2_hbm, out_hbm,
             idx_all, fbufs, abufs, bbufs, cbufs, res_v, obufs,
             gsems, osems):
    wid = lax.axis_index("s") * 2 + lax.axis_index("c")
    base = wid * _PER_W
    obase = wid * _BAGS_W * _D
    lanes = lax.iota(jnp.int32, 16)
    pats = [lanes * 8 + r2 for r2 in range(8)]

    pltpu.sync_copy(idx_hbm.at[pl.ds(base, _PER_W)], idx_all)

    def fire(g, s):
        f0_v, f1_v, f2_v = fbufs[s]
        for v in range(_CHUNK // 16):
            ii = idx_all[pl.ds(g * _CHUNK + v * 16, 16)]
            t100 = ((base + g * _CHUNK + v * 16 + lanes) // (_B * _L)) * _P[0]
            f0_v[pl.ds(v * 16, 16)] = t100 + ii // (_P[1] * _P[2])
            f1_v[pl.ds(v * 16, 16)] = t100 + (ii // _P[2]) % _P[1]
            f2_v[pl.ds(v * 16, 16)] = t100 + ii % _P[2]
        pltpu.async_copy(c0_hbm.at[f0_v], abufs[s], gsems[s][0])
        pltpu.async_copy(c1_hbm.at[f1_v], bbufs[s], gsems[s][1])
        pltpu.async_copy(c2_hbm.at[f2_v], cbufs[s], gsems[s][2])

    def compute(g, h, s):
        f0_v, _, _ = fbufs[s]
        a_v, b_v, c_v, ob_v = abufs[s], bbufs[s], cbufs[s], obufs[s]
        pltpu.make_async_copy(c0_hbm.at[f0_v], a_v, gsems[s][0]).wait()
        pltpu.make_async_copy(c1_hbm.at[f0_v], b_v, gsems[s][1]).wait()
        pltpu.make_async_copy(c2_hbm.at[f0_v], c_v, gsems[s][2]).wait()
        out_slice = out_hbm.at[pl.ds(obase + g * _BPC * _D, _BPC * _D)]
        pl.when(h > 0)(
            lambda: pltpu.make_async_copy(ob_v, out_slice, osems[s]).wait())

        def bag_body(bb, carry2):
            def lk_body(l, accs):
                li = bb * _L + l
                av = [a_v[li, pl.ds(16 * k, 16)] for k in range(2)]
                cv = [c_v[li, pl.ds(16 * k, 16)] for k in range(2)]
                bv = [b_v[li, pl.ds(16 * k, 16)] for k in range(16)]
                for j in range(8):
                    q0, h2 = j // 2, j % 2
                    r = av[q0 // 2][(q0 * 8) % 16] * bv[h2]
                    for r1 in range(1, 8):
                        p = q0 * 8 + r1
                        r = r + av[p // 16][p % 16] * bv[r1 * 2 + h2]
                    res_v[pl.ds(16 * j, 16)] = r
                a0, a1, a2, a3 = accs
                for r2 in range(8):
                    gv = plsc.load_gather(res_v, [pats[r2]])
                    a0 = a0 + gv * cv[(r2 * 4) // 16][(r2 * 4) % 16]
                    a1 = a1 + gv * cv[(r2 * 4 + 1) // 16][(r2 * 4 + 1) % 16]
                    a2 = a2 + gv * cv[(r2 * 4 + 2) // 16][(r2 * 4 + 2) % 16]
                    a3 = a3 + gv * cv[(r2 * 4 + 3) // 16][(r2 * 4 + 3) % 16]
                return (a0, a1, a2, a3)

            z = jnp.zeros((16,), jnp.float32)
            accs = lax.fori_loop(0, _L, lk_body, (z, z, z, z))
            for m in range(4):
                plsc.store_scatter(ob_v, [bb * _D + lanes * 4 + m], accs[m])
            return carry2

        lax.fori_loop(0, _BPC, bag_body, 0)
        pltpu.async_copy(ob_v, out_slice, osems[s])

    fire(0, 0)
    fire(1, 1)

    def loop_body(h, carry):
        compute(2 * h, h, 0)
        pl.when(h < _NCHUNK // 2 - 1)(lambda: fire(2 * h + 2, 0))
        compute(2 * h + 1, h, 1)
        pl.when(h < _NCHUNK // 2 - 1)(lambda: fire(2 * h + 3, 1))
        return carry

    lax.fori_loop(0, _NCHUNK // 2, loop_body, 0)
    tail = out_hbm.at[pl.ds(obase, _BPC * _D)]
    pltpu.make_async_copy(obufs[0], tail, osems[0]).wait()
    pltpu.make_async_copy(obufs[1], tail, osems[1]).wait()


@jax.jit
def _tt_bag_sc(indices, c0f, c1f, c2f):
    mesh = plsc.VectorSubcoreMesh(core_axis_name="c", subcore_axis_name="s")
    f = pl.kernel(
        _sc_body,
        out_type=jax.ShapeDtypeStruct((_T * _B * _D,), jnp.float32),
        mesh=mesh,
        compiler_params=pltpu.CompilerParams(needs_layout_passes=False,
                                             use_tc_tiling_on_sc=False),
        scratch_types=[
            pltpu.VMEM((_PER_W,), jnp.int32),
            [[pltpu.VMEM((_CHUNK,), jnp.int32) for _ in range(3)]
             for _ in range(2)],
            [pltpu.VMEM((_CHUNK, 32), jnp.float32) for _ in range(2)],
            [pltpu.VMEM((_CHUNK, 256), jnp.float32) for _ in range(2)],
            [pltpu.VMEM((_CHUNK, 32), jnp.float32) for _ in range(2)],
            pltpu.VMEM((128,), jnp.float32),
            [pltpu.VMEM((_BPC * _D,), jnp.float32) for _ in range(2)],
            [[pltpu.SemaphoreType.DMA for _ in range(3)] for _ in range(2)],
            [pltpu.SemaphoreType.DMA for _ in range(2)],
        ],
    )
    return f(indices, c0f, c1f, c2f)


def kernel(indices, offsets, tt_core0, tt_core1, tt_core2):
    del offsets
    c0f = tt_core0.reshape(_T * _P[0], 32)
    c1f = tt_core1.reshape(_T * _P[1], 256)
    c2f = tt_core2.reshape(_T * _P[2], 32)
    out = _tt_bag_sc(indices, c0f, c1f, c2f)
    return out.reshape(_T * _B, _D)


# speedup vs baseline: 84.6791x; 1.1246x over previous
import jax
import jax.numpy as jnp
from jax import lax
from jax.experimental import pallas as pl
from jax.experimental.pallas import tpu as pltpu, tpu_sc as plsc

_T = 26
_B = 1024
_L = 20
_D = 64
_P = (100, 100, 100)
_NNZ = _T * _B * _L
_NW = 32
_PER_W = _NNZ // _NW
_BAGS_W = _PER_W // _L
_CHUNK = 80
_NCHUNK = _PER_W // _CHUNK
_BPC = _CHUNK // _L


def _sc_body(idx_hbm, c0_hbm, c1_hbm, c2_hbm, out_hbm,
             idx_all, fbufs, abufs, bbufs, cbufs, res_v, obufs,
             gsems, osems):
    wid = lax.axis_index("s") * 2 + lax.axis_index("c")
    base = wid * _PER_W
    obase = wid * _BAGS_W * _D
    lanes = lax.iota(jnp.int32, 16)
    pats = [lanes * 8 + r2 for r2 in range(8)]

    pltpu.sync_copy(idx_hbm.at[pl.ds(base, _PER_W)], idx_all)

    def fire(g, s):
        f0_v, f1_v, f2_v = fbufs[s]
        t100 = ((base + g * _CHUNK) // (_B * _L)) * _P[0]
        inv = jnp.float32(0.01)
        eps = jnp.float32(0.005)
        for v in range(_CHUNK // 16):
            ii = idx_all[pl.ds(g * _CHUNK + v * 16, 16)]
            u = (ii.astype(jnp.float32) * inv + eps).astype(jnp.int32)
            i0 = (u.astype(jnp.float32) * inv + eps).astype(jnp.int32)
            f0_v[pl.ds(v * 16, 16)] = t100 + i0
            f1_v[pl.ds(v * 16, 16)] = t100 + (u - i0 * _P[1])
            f2_v[pl.ds(v * 16, 16)] = t100 + (ii - u * _P[2])
        pltpu.async_copy(c0_hbm.at[f0_v], abufs[s], gsems[s][0])
        pltpu.async_copy(c1_hbm.at[f1_v], bbufs[s], gsems[s][1])
        pltpu.async_copy(c2_hbm.at[f2_v], cbufs[s], gsems[s][2])

    def compute(g, h, s):
        f0_v, _, _ = fbufs[s]
        a_v, b_v, c_v, ob_v = abufs[s], bbufs[s], cbufs[s], obufs[s]
        pltpu.make_async_copy(c0_hbm.at[f0_v], a_v, gsems[s][0]).wait()
        pltpu.make_async_copy(c1_hbm.at[f0_v], b_v, gsems[s][1]).wait()
        pltpu.make_async_copy(c2_hbm.at[f0_v], c_v, gsems[s][2]).wait()
        out_slice = out_hbm.at[pl.ds(obase + g * _BPC * _D, _BPC * _D)]
        pl.when(h > 0)(
            lambda: pltpu.make_async_copy(ob_v, out_slice, osems[s]).wait())

        def bag_body(bb, carry2):
            def lk_body(l, accs):
                li = bb * _L + l
                av = [a_v[li, pl.ds(16 * k, 16)] for k in range(2)]
                cv = [c_v[li, pl.ds(16 * k, 16)] for k in range(2)]
                bv = [b_v[li, pl.ds(16 * k, 16)] for k in range(16)]
                for j in range(8):
                    q0, h2 = j // 2, j % 2
                    r = av[q0 // 2][(q0 * 8) % 16] * bv[h2]
                    for r1 in range(1, 8):
                        p = q0 * 8 + r1
                        r = r + av[p // 16][p % 16] * bv[r1 * 2 + h2]
                    res_v[pl.ds(16 * j, 16)] = r
                a0, a1, a2, a3 = accs
                for r2 in range(8):
                    gv = plsc.load_gather(res_v, [pats[r2]])
                    a0 = a0 + gv * cv[(r2 * 4) // 16][(r2 * 4) % 16]
                    a1 = a1 + gv * cv[(r2 * 4 + 1) // 16][(r2 * 4 + 1) % 16]
                    a2 = a2 + gv * cv[(r2 * 4 + 2) // 16][(r2 * 4 + 2) % 16]
                    a3 = a3 + gv * cv[(r2 * 4 + 3) // 16][(r2 * 4 + 3) % 16]
                return (a0, a1, a2, a3)

            z = jnp.zeros((16,), jnp.float32)
            accs = lax.fori_loop(0, _L, lk_body, (z, z, z, z))
            for m in range(4):
                plsc.store_scatter(ob_v, [bb * _D + lanes * 4 + m], accs[m])
            return carry2

        lax.fori_loop(0, _BPC, bag_body, 0)
        pltpu.async_copy(ob_v, out_slice, osems[s])

    fire(0, 0)
    fire(1, 1)

    def loop_body(h, carry):
        compute(2 * h, h, 0)
        pl.when(h < _NCHUNK // 2 - 1)(lambda: fire(2 * h + 2, 0))
        compute(2 * h + 1, h, 1)
        pl.when(h < _NCHUNK // 2 - 1)(lambda: fire(2 * h + 3, 1))
        return carry

    lax.fori_loop(0, _NCHUNK // 2, loop_body, 0)
    tail = out_hbm.at[pl.ds(obase, _BPC * _D)]
    pltpu.make_async_copy(obufs[0], tail, osems[0]).wait()
    pltpu.make_async_copy(obufs[1], tail, osems[1]).wait()


@jax.jit
def _tt_bag_sc(indices, c0f, c1f, c2f):
    mesh = plsc.VectorSubcoreMesh(core_axis_name="c", subcore_axis_name="s")
    f = pl.kernel(
        _sc_body,
        out_type=jax.ShapeDtypeStruct((_T * _B * _D,), jnp.float32),
        mesh=mesh,
        compiler_params=pltpu.CompilerParams(needs_layout_passes=False,
                                             use_tc_tiling_on_sc=False),
        scratch_types=[
            pltpu.VMEM((_PER_W,), jnp.int32),
            [[pltpu.VMEM((_CHUNK,), jnp.int32) for _ in range(3)]
             for _ in range(2)],
            [pltpu.VMEM((_CHUNK, 32), jnp.float32) for _ in range(2)],
            [pltpu.VMEM((_CHUNK, 256), jnp.float32) for _ in range(2)],
            [pltpu.VMEM((_CHUNK, 32), jnp.float32) for _ in range(2)],
            pltpu.VMEM((128,), jnp.float32),
            [pltpu.VMEM((_BPC * _D,), jnp.float32) for _ in range(2)],
            [[pltpu.SemaphoreType.DMA for _ in range(3)] for _ in range(2)],
            [pltpu.SemaphoreType.DMA for _ in range(2)],
        ],
    )
    return f(indices, c0f, c1f, c2f)


def kernel(indices, offsets, tt_core0, tt_core1, tt_core2):
    del offsets
    c0f = tt_core0.reshape(_T * _P[0], 32)
    c1f = tt_core1.reshape(_T * _P[1], 256)
    c2f = tt_core2.reshape(_T * _P[2], 32)
    out = _tt_bag_sc(indices, c0f, c1f, c2f)
    return out.reshape(_T * _B, _D)


# speedup vs baseline: 93.3194x; 1.1020x over previous
import jax
import jax.numpy as jnp
from jax import lax
from jax.experimental import pallas as pl
from jax.experimental.pallas import tpu as pltpu, tpu_sc as plsc

_T = 26
_B = 1024
_L = 20
_D = 64
_P = (100, 100, 100)
_NNZ = _T * _B * _L
_NW = 32
_PER_W = _NNZ // _NW
_BAGS_W = _PER_W // _L
_CHUNK = 80
_NCHUNK = _PER_W // _CHUNK
_BPC = _CHUNK // _L


def _sc_body(idx_hbm, c0_hbm, c1_hbm, c2_hbm, out_hbm,
             idx_all, fbufs, abufs, bbufs, cbufs, res_v, obufs,
             gsems, osems):
    wid = lax.axis_index("s") * 2 + lax.axis_index("c")
    base = wid * _PER_W
    obase = wid * _BAGS_W * _D
    lanes = lax.iota(jnp.int32, 16)
    pats = [lanes * 8 + r2 for r2 in range(8)]

    pltpu.sync_copy(idx_hbm.at[pl.ds(base, _PER_W)], idx_all)

    def fire(g, s):
        f0_v, f1_v, f2_v = fbufs[s]
        t100 = ((base + g * _CHUNK) // (_B * _L)) * _P[0]
        inv = jnp.float32(0.01)
        eps = jnp.float32(0.005)
        for v in range(_CHUNK // 16):
            ii = idx_all[pl.ds(g * _CHUNK + v * 16, 16)]
            u = (ii.astype(jnp.float32) * inv + eps).astype(jnp.int32)
            i0 = (u.astype(jnp.float32) * inv + eps).astype(jnp.int32)
            f0_v[pl.ds(v * 16, 16)] = t100 + i0
            f1_v[pl.ds(v * 16, 16)] = t100 + (u - i0 * _P[1])
            f2_v[pl.ds(v * 16, 16)] = t100 + (ii - u * _P[2])
        pltpu.async_copy(c0_hbm.at[f0_v], abufs[s], gsems[s][0])
        pltpu.async_copy(c1_hbm.at[f1_v], bbufs[s], gsems[s][1])
        pltpu.async_copy(c2_hbm.at[f2_v], cbufs[s], gsems[s][2])

    def compute(g, h, s):
        f0_v, _, _ = fbufs[s]
        a_v, b_v, c_v, ob_v = abufs[s], bbufs[s], cbufs[s], obufs[s]
        pltpu.make_async_copy(c0_hbm.at[f0_v], a_v, gsems[s][0]).wait()
        pltpu.make_async_copy(c1_hbm.at[f0_v], b_v, gsems[s][1]).wait()
        pltpu.make_async_copy(c2_hbm.at[f0_v], c_v, gsems[s][2]).wait()
        out_slice = out_hbm.at[pl.ds(obase + g * _BPC * _D, _BPC * _D)]
        pl.when(h > 0)(
            lambda: pltpu.make_async_copy(ob_v, out_slice, osems[s]).wait())

        def bag_body(bb, carry2):
            def one_lookup(li, rbuf, accs):
                av = [a_v[li, pl.ds(16 * k, 16)] for k in range(2)]
                cv = [c_v[li, pl.ds(16 * k, 16)] for k in range(2)]
                bv = [b_v[li, pl.ds(16 * k, 16)] for k in range(16)]
                for j in range(8):
                    q0, h2 = j // 2, j % 2
                    r = av[q0 // 2][(q0 * 8) % 16] * bv[h2]
                    for r1 in range(1, 8):
                        p = q0 * 8 + r1
                        r = r + av[p // 16][p % 16] * bv[r1 * 2 + h2]
                    rbuf[pl.ds(16 * j, 16)] = r
                a0, a1, a2, a3 = accs
                for r2 in range(8):
                    gv = plsc.load_gather(rbuf, [pats[r2]])
                    a0 = a0 + gv * cv[(r2 * 4) // 16][(r2 * 4) % 16]
                    a1 = a1 + gv * cv[(r2 * 4 + 1) // 16][(r2 * 4 + 1) % 16]
                    a2 = a2 + gv * cv[(r2 * 4 + 2) // 16][(r2 * 4 + 2) % 16]
                    a3 = a3 + gv * cv[(r2 * 4 + 3) // 16][(r2 * 4 + 3) % 16]
                return (a0, a1, a2, a3)

            def lk_body(l, accs):
                li = bb * _L + 2 * l
                accs = one_lookup(li, res_v[0], accs)
                return one_lookup(li + 1, res_v[1], accs)

            z = jnp.zeros((16,), jnp.float32)
            accs = lax.fori_loop(0, _L // 2, lk_body, (z, z, z, z))
            for m in range(4):
                plsc.store_scatter(ob_v, [bb * _D + lanes * 4 + m], accs[m])
            return carry2

        lax.fori_loop(0, _BPC, bag_body, 0)
        pltpu.async_copy(ob_v, out_slice, osems[s])

    fire(0, 0)
    fire(1, 1)

    def loop_body(h, carry):
        compute(2 * h, h, 0)
        pl.when(h < _NCHUNK // 2 - 1)(lambda: fire(2 * h + 2, 0))
        compute(2 * h + 1, h, 1)
        pl.when(h < _NCHUNK // 2 - 1)(lambda: fire(2 * h + 3, 1))
        return carry

    lax.fori_loop(0, _NCHUNK // 2, loop_body, 0)
    tail = out_hbm.at[pl.ds(obase, _BPC * _D)]
    pltpu.make_async_copy(obufs[0], tail, osems[0]).wait()
    pltpu.make_async_copy(obufs[1], tail, osems[1]).wait()


@jax.jit
def _tt_bag_sc(indices, c0f, c1f, c2f):
    mesh = plsc.VectorSubcoreMesh(core_axis_name="c", subcore_axis_name="s")
    f = pl.kernel(
        _sc_body,
        out_type=jax.ShapeDtypeStruct((_T * _B * _D,), jnp.float32),
        mesh=mesh,
        compiler_params=pltpu.CompilerParams(needs_layout_passes=False,
                                             use_tc_tiling_on_sc=False),
        scratch_types=[
            pltpu.VMEM((_PER_W,), jnp.int32),
            [[pltpu.VMEM((_CHUNK,), jnp.int32) for _ in range(3)]
             for _ in range(2)],
            [pltpu.VMEM((_CHUNK, 32), jnp.float32) for _ in range(2)],
            [pltpu.VMEM((_CHUNK, 256), jnp.float32) for _ in range(2)],
            [pltpu.VMEM((_CHUNK, 32), jnp.float32) for _ in range(2)],
            [pltpu.VMEM((128,), jnp.float32) for _ in range(2)],
            [pltpu.VMEM((_BPC * _D,), jnp.float32) for _ in range(2)],
            [[pltpu.SemaphoreType.DMA for _ in range(3)] for _ in range(2)],
            [pltpu.SemaphoreType.DMA for _ in range(2)],
        ],
    )
    return f(indices, c0f, c1f, c2f)


def kernel(indices, offsets, tt_core0, tt_core1, tt_core2):
    del offsets
    c0f = tt_core0.reshape(_T * _P[0], 32)
    c1f = tt_core1.reshape(_T * _P[1], 256)
    c2f = tt_core2.reshape(_T * _P[2], 32)
    out = _tt_bag_sc(indices, c0f, c1f, c2f)
    return out.reshape(_T * _B, _D)
